# baseline (speedup 1.0000x reference)
import functools

import jax
import jax.numpy as jnp
from jax import lax
from jax.experimental import pallas as pl
from jax.experimental.pallas import tpu as pltpu
from jax.experimental.pallas import tpu_sc as plsc

_N = 10000
_D = 128
_E = 320000
_NC = 2
_NS = 16
_NW = _NC * _NS
_CH = 128
_EPAD = 327680
_EPW = _EPAD // _NW
_NCHUNK = _EPW // _CH
_NPAD = 10240
_RPS = _NPAD // _NS
_DEGW = 128

def _mesh():
    return plsc.VectorSubcoreMesh(core_axis_name="c", subcore_axis_name="s")


def _sc_degree(dst, ones_hbm_v, zeros_hbm_v):
    @functools.partial(
        pl.kernel,
        out_type=jax.ShapeDtypeStruct((_NC * _NPAD, _DEGW), jnp.float32),
        mesh=_mesh(),
        scratch_types=[
            pltpu.VMEM((_CH,), jnp.int32),
            pltpu.VMEM((_CH, _DEGW), jnp.float32),
            pltpu.VMEM_SHARED((_NPAD, _DEGW), jnp.float32),
        ],
    )
    def k(dst_hbm, ones_hbm, zeros_hbm, out_hbm, didx, ones_v, acc):
        c = lax.axis_index("c")
        s = lax.axis_index("s")
        wid = s * _NC + c

        pltpu.sync_copy(ones_hbm, ones_v)
        pltpu.sync_copy(zeros_hbm, acc.at[pl.ds(s * _RPS, _RPS)])

        plsc.subcore_barrier()

        @pl.loop(0, _NCHUNK)
        def _(ci):
            pltpu.sync_copy(dst_hbm.at[pl.ds(wid * _EPW + ci * _CH, _CH)], didx)
            pltpu.sync_copy(ones_v, acc.at[didx], add=True)

        plsc.subcore_barrier()

        @pl.loop(0, _RPS, step=_CH)
        def _(r0):
            r = s * _RPS + r0
            pltpu.sync_copy(acc.at[pl.ds(r, _CH)],
                            out_hbm.at[pl.ds(c * _NPAD + r, _CH)])

    return k(dst, ones_hbm_v, zeros_hbm_v)


def _sc_scatter_accum(y, src, dst, zeros_hbm_v):
    @functools.partial(
        pl.kernel,
        out_type=jax.ShapeDtypeStruct((_NC * _NPAD, _D), jnp.float32),
        mesh=_mesh(),
        scratch_types=[
            pltpu.VMEM((_CH,), jnp.int32),
            pltpu.VMEM((_CH,), jnp.int32),
            pltpu.VMEM((_CH,), jnp.int32),
            pltpu.VMEM((_CH,), jnp.int32),
            pltpu.VMEM((_CH, _D), jnp.float32),
            pltpu.VMEM((_CH, _D), jnp.float32),
            pltpu.VMEM_SHARED((_NPAD, _D), jnp.float32),
            pltpu.SemaphoreType.DMA,
            pltpu.SemaphoreType.DMA,
        ],
    )
    def k(y_hbm, src_hbm, dst_hbm, zeros_hbm, out_hbm,
          sidx0, sidx1, didx0, didx1, rows0, rows1, acc, gsem0, gsem1):
        c = lax.axis_index("c")
        s = lax.axis_index("s")
        wid = s * _NC + c

        pltpu.sync_copy(zeros_hbm, acc.at[pl.ds(s * _RPS, _RPS)])

        plsc.subcore_barrier()

        @pl.loop(0, _NCHUNK)
        def _(ci):
            base = wid * _EPW + ci * _CH
            pltpu.sync_copy(src_hbm.at[pl.ds(base, _CH)], sidx0)
            pltpu.async_copy(y_hbm.at[sidx0], rows0, gsem0).wait()
            pltpu.sync_copy(dst_hbm.at[pl.ds(base, _CH)], didx0)
            pltpu.sync_copy(rows0, acc.at[didx0], add=True)

        plsc.subcore_barrier()

        @pl.loop(0, _RPS, step=_CH)
        def _(r0):
            r = s * _RPS + r0
            pltpu.sync_copy(acc.at[pl.ds(r, _CH)],
                            out_hbm.at[pl.ds(c * _NPAD + r, _CH)])

    return k(y, src, dst, zeros_hbm_v)


def _dis_expr(degp_ref):
    deg = degp_ref[0:_N, 0:1] + degp_ref[_NPAD:_NPAD + _N, 0:1] + 1.0
    return lax.rsqrt(deg)


def _tc_matmul(x, w):
    def body(x_ref, w_ref, o_ref):
        o_ref[...] = jnp.dot(x_ref[...], w_ref[...],
                             preferred_element_type=jnp.float32)

    return pl.pallas_call(
        body,
        out_shape=jax.ShapeDtypeStruct((x.shape[0], w.shape[1]), jnp.float32),
    )(x, w)


def _tc_scale(degp, xw):
    def body(degp_ref, xw_ref, o_ref):
        o_ref[...] = xw_ref[...] * _dis_expr(degp_ref)

    return pl.pallas_call(
        body,
        out_shape=jax.ShapeDtypeStruct((_N, _D), jnp.float32),
    )(degp, xw)


def _tc_mid(accp, y1, degp, b1, w2):
    def body(acc_ref, y1_ref, degp_ref, b1_ref, w2_ref, o_ref):
        dis = _dis_expr(degp_ref)
        ssum = acc_ref[0:_N, :] + acc_ref[_NPAD:_NPAD + _N, :] + y1_ref[...]
        h = jnp.maximum(ssum * dis + b1_ref[...], 0.0)
        o_ref[...] = jnp.dot(h, w2_ref[...],
                             preferred_element_type=jnp.float32) * dis

    return pl.pallas_call(
        body,
        out_shape=jax.ShapeDtypeStruct((_N, _D), jnp.float32),
    )(accp, y1, degp, b1, w2)


def _tc_out(accp, y2, degp, b2):
    def body(acc_ref, y2_ref, degp_ref, b2_ref, o_ref):
        dis = _dis_expr(degp_ref)
        z = (acc_ref[0:_N, :] + acc_ref[_NPAD:_NPAD + _N, :] + y2_ref[...]) \
            * dis + b2_ref[...]
        m = jnp.max(z, axis=1, keepdims=True)
        e = jnp.exp(z - m)
        o_ref[...] = (z - m) - jnp.log(jnp.sum(e, axis=1, keepdims=True))

    return pl.pallas_call(
        body,
        out_shape=jax.ShapeDtypeStruct((_N, _D), jnp.float32),
    )(accp, y2, degp, b2)


def kernel(feature, edge_index, use_feature, W1, b1, W2, b2):
    x = jnp.where(use_feature != 0, feature,
                  jnp.eye(_N, _D, dtype=feature.dtype))
    ei = edge_index.astype(jnp.int32)
    pad = _EPAD - _E
    src = jnp.concatenate([ei[0], jnp.zeros((pad,), jnp.int32)])
    dst = jnp.concatenate([ei[1], jnp.full((pad,), _N, jnp.int32)])
    b1r = b1.reshape(1, _D).astype(jnp.float32)
    b2r = b2.reshape(1, _D).astype(jnp.float32)
    ones_deg = jnp.ones((_CH, _DEGW), jnp.float32)
    zeros_deg = jnp.zeros((_RPS, _DEGW), jnp.float32)
    zeros_acc = jnp.zeros((_RPS, _D), jnp.float32)

    degp = _sc_degree(dst, ones_deg, zeros_deg)
    xw1 = _tc_matmul(x, W1)
    y1 = _tc_scale(degp, xw1)
    acc1 = _sc_scatter_accum(y1, src, dst, zeros_acc)
    y2 = _tc_mid(acc1, y1, degp, b1r, W2)
    acc2 = _sc_scatter_accum(y2, src, dst, zeros_acc)
    return _tc_out(acc2, y2, degp, b2r)


# speedup vs baseline: 1.2473x; 1.2473x over previous
import functools

import jax
import jax.numpy as jnp
from jax import lax
from jax.experimental import pallas as pl
from jax.experimental.pallas import tpu as pltpu
from jax.experimental.pallas import tpu_sc as plsc

_N = 10000
_D = 128
_E = 320000
_NC = 2
_NS = 16
_NW = _NC * _NS
_CH = 128
_EPAD = 327680
_EPW = _EPAD // _NW
_NCHUNK = _EPW // _CH
_NPAD = 10240
_RPS = _NPAD // _NS
_DEGW = 128

def _mesh():
    return plsc.VectorSubcoreMesh(core_axis_name="c", subcore_axis_name="s")


def _sc_degree(dst, ones_hbm_v, zeros_hbm_v):
    @functools.partial(
        pl.kernel,
        out_type=jax.ShapeDtypeStruct((_NC * _NPAD, _DEGW), jnp.float32),
        mesh=_mesh(),
        scratch_types=[
            pltpu.VMEM((_CH,), jnp.int32),
            pltpu.VMEM((_CH, _DEGW), jnp.float32),
            pltpu.VMEM_SHARED((_NPAD, _DEGW), jnp.float32),
        ],
    )
    def k(dst_hbm, ones_hbm, zeros_hbm, out_hbm, didx, ones_v, acc):
        c = lax.axis_index("c")
        s = lax.axis_index("s")
        wid = s * _NC + c

        pltpu.sync_copy(ones_hbm, ones_v)
        pltpu.sync_copy(zeros_hbm, acc.at[pl.ds(s * _RPS, _RPS)])

        plsc.subcore_barrier()

        @pl.loop(0, _NCHUNK)
        def _(ci):
            pltpu.sync_copy(dst_hbm.at[pl.ds(wid * _EPW + ci * _CH, _CH)], didx)
            pltpu.sync_copy(ones_v, acc.at[didx], add=True)

        plsc.subcore_barrier()

        @pl.loop(0, _RPS, step=_CH)
        def _(r0):
            r = s * _RPS + r0
            pltpu.sync_copy(acc.at[pl.ds(r, _CH)],
                            out_hbm.at[pl.ds(c * _NPAD + r, _CH)])

    return k(dst, ones_hbm_v, zeros_hbm_v)


def _sc_scatter_accum(y, src, dst, zeros_hbm_v):
    @functools.partial(
        pl.kernel,
        out_type=jax.ShapeDtypeStruct((_NC * _NPAD, _D), jnp.float32),
        mesh=_mesh(),
        scratch_types=[
            pltpu.VMEM((_CH,), jnp.int32),
            pltpu.VMEM((_CH,), jnp.int32),
            pltpu.VMEM((_CH,), jnp.int32),
            pltpu.VMEM((_CH,), jnp.int32),
            pltpu.VMEM((_CH, _D), jnp.float32),
            pltpu.VMEM((_CH, _D), jnp.float32),
            pltpu.VMEM_SHARED((_NPAD, _D), jnp.float32),
            pltpu.SemaphoreType.DMA,
            pltpu.SemaphoreType.DMA,
        ],
    )
    def k(y_hbm, src_hbm, dst_hbm, zeros_hbm, out_hbm,
          sidx0, sidx1, didx0, didx1, rows0, rows1, acc, gsem0, gsem1):
        c = lax.axis_index("c")
        s = lax.axis_index("s")
        wid = s * _NC + c

        pltpu.sync_copy(zeros_hbm, acc.at[pl.ds(s * _RPS, _RPS)])

        plsc.subcore_barrier()

        def fire(ci, sidx, rows, gsem):
            pltpu.sync_copy(src_hbm.at[pl.ds(wid * _EPW + ci * _CH, _CH)], sidx)
            pltpu.async_copy(y_hbm.at[sidx], rows, gsem)

        def drain(sidx, rows, gsem):
            pltpu.make_async_copy(y_hbm.at[sidx], rows, gsem).wait()

        def scat(ci, didx, rows):
            pltpu.sync_copy(dst_hbm.at[pl.ds(wid * _EPW + ci * _CH, _CH)], didx)
            pltpu.sync_copy(rows, acc.at[didx], add=True)

        fire(0, sidx0, rows0, gsem0)
        fire(1, sidx1, rows1, gsem1)

        @pl.loop(0, _NCHUNK - 2, step=2)
        def _(ci):
            drain(sidx0, rows0, gsem0)
            scat(ci, didx0, rows0)
            fire(ci + 2, sidx0, rows0, gsem0)
            drain(sidx1, rows1, gsem1)
            scat(ci + 1, didx1, rows1)
            fire(ci + 3, sidx1, rows1, gsem1)

        drain(sidx0, rows0, gsem0)
        scat(_NCHUNK - 2, didx0, rows0)
        drain(sidx1, rows1, gsem1)
        scat(_NCHUNK - 1, didx1, rows1)

        plsc.subcore_barrier()

        @pl.loop(0, _RPS, step=_CH)
        def _(r0):
            r = s * _RPS + r0
            pltpu.sync_copy(acc.at[pl.ds(r, _CH)],
                            out_hbm.at[pl.ds(c * _NPAD + r, _CH)])

    return k(y, src, dst, zeros_hbm_v)


def _dis_expr(degp_ref):
    deg = degp_ref[0:_N, 0:1] + degp_ref[_NPAD:_NPAD + _N, 0:1] + 1.0
    return lax.rsqrt(deg)


def _tc_matmul(x, w):
    def body(x_ref, w_ref, o_ref):
        o_ref[...] = jnp.dot(x_ref[...], w_ref[...],
                             preferred_element_type=jnp.float32)

    return pl.pallas_call(
        body,
        out_shape=jax.ShapeDtypeStruct((x.shape[0], w.shape[1]), jnp.float32),
    )(x, w)


def _tc_scale(degp, xw):
    def body(degp_ref, xw_ref, o_ref):
        o_ref[...] = xw_ref[...] * _dis_expr(degp_ref)

    return pl.pallas_call(
        body,
        out_shape=jax.ShapeDtypeStruct((_N, _D), jnp.float32),
    )(degp, xw)


def _tc_mid(accp, y1, degp, b1, w2):
    def body(acc_ref, y1_ref, degp_ref, b1_ref, w2_ref, o_ref):
        dis = _dis_expr(degp_ref)
        ssum = acc_ref[0:_N, :] + acc_ref[_NPAD:_NPAD + _N, :] + y1_ref[...]
        h = jnp.maximum(ssum * dis + b1_ref[...], 0.0)
        o_ref[...] = jnp.dot(h, w2_ref[...],
                             preferred_element_type=jnp.float32) * dis

    return pl.pallas_call(
        body,
        out_shape=jax.ShapeDtypeStruct((_N, _D), jnp.float32),
    )(accp, y1, degp, b1, w2)


def _tc_out(accp, y2, degp, b2):
    def body(acc_ref, y2_ref, degp_ref, b2_ref, o_ref):
        dis = _dis_expr(degp_ref)
        z = (acc_ref[0:_N, :] + acc_ref[_NPAD:_NPAD + _N, :] + y2_ref[...]) \
            * dis + b2_ref[...]
        m = jnp.max(z, axis=1, keepdims=True)
        e = jnp.exp(z - m)
        o_ref[...] = (z - m) - jnp.log(jnp.sum(e, axis=1, keepdims=True))

    return pl.pallas_call(
        body,
        out_shape=jax.ShapeDtypeStruct((_N, _D), jnp.float32),
    )(accp, y2, degp, b2)


def kernel(feature, edge_index, use_feature, W1, b1, W2, b2):
    x = jnp.where(use_feature != 0, feature,
                  jnp.eye(_N, _D, dtype=feature.dtype))
    ei = edge_index.astype(jnp.int32)
    pad = _EPAD - _E
    src = jnp.concatenate([ei[0], jnp.zeros((pad,), jnp.int32)])
    trash = _N + jnp.arange(pad, dtype=jnp.int32) % (_NPAD - _N)
    dst = jnp.concatenate([ei[1], trash])
    b1r = b1.reshape(1, _D).astype(jnp.float32)
    b2r = b2.reshape(1, _D).astype(jnp.float32)
    ones_deg = jnp.ones((_CH, _DEGW), jnp.float32)
    zeros_deg = jnp.zeros((_RPS, _DEGW), jnp.float32)
    zeros_acc = jnp.zeros((_RPS, _D), jnp.float32)

    degp = _sc_degree(dst, ones_deg, zeros_deg)
    xw1 = _tc_matmul(x, W1)
    y1 = _tc_scale(degp, xw1)
    acc1 = _sc_scatter_accum(y1, src, dst, zeros_acc)
    y2 = _tc_mid(acc1, y1, degp, b1r, W2)
    acc2 = _sc_scatter_accum(y2, src, dst, zeros_acc)
    return _tc_out(acc2, y2, degp, b2r)


# speedup vs baseline: 1.2733x; 1.0208x over previous
import functools

import jax
import jax.numpy as jnp
from jax import lax
from jax.experimental import pallas as pl
from jax.experimental.pallas import tpu as pltpu
from jax.experimental.pallas import tpu_sc as plsc

_N = 10000
_D = 128
_E = 320000
_NC = 2
_NS = 16
_NW = _NC * _NS
_CH = 128
_EPAD = 327680
_EPW = _EPAD // _NW
_NCHUNK = _EPW // _CH
_NPAD = 10240
_RPS = _NPAD // _NS
_DEGW = 128

def _mesh():
    return plsc.VectorSubcoreMesh(core_axis_name="c", subcore_axis_name="s")


def _sc_degree(dst, ones_hbm_v, zeros_hbm_v):
    @functools.partial(
        pl.kernel,
        out_type=jax.ShapeDtypeStruct((_NC * _NPAD, _DEGW), jnp.float32),
        mesh=_mesh(),
        scratch_types=[
            pltpu.VMEM((_CH,), jnp.int32),
            pltpu.VMEM((_CH, _DEGW), jnp.float32),
            pltpu.VMEM_SHARED((_NPAD, _DEGW), jnp.float32),
        ],
    )
    def k(dst_hbm, ones_hbm, zeros_hbm, out_hbm, didx, ones_v, acc):
        c = lax.axis_index("c")
        s = lax.axis_index("s")
        wid = s * _NC + c

        pltpu.sync_copy(ones_hbm, ones_v)
        pltpu.sync_copy(zeros_hbm, acc.at[pl.ds(s * _RPS, _RPS)])

        plsc.subcore_barrier()

        @pl.loop(0, _NCHUNK)
        def _(ci):
            pltpu.sync_copy(dst_hbm.at[pl.ds(wid * _EPW + ci * _CH, _CH)], didx)
            pltpu.sync_copy(ones_v, acc.at[didx], add=True)

        plsc.subcore_barrier()

        @pl.loop(0, _RPS, step=_CH)
        def _(r0):
            r = s * _RPS + r0
            pltpu.sync_copy(acc.at[pl.ds(r, _CH)],
                            out_hbm.at[pl.ds(c * _NPAD + r, _CH)])

    return k(dst, ones_hbm_v, zeros_hbm_v)


def _sc_scatter_accum(y, src, dst, zeros_hbm_v):
    @functools.partial(
        pl.kernel,
        out_type=jax.ShapeDtypeStruct((_NC * _NPAD, _D), jnp.float32),
        mesh=_mesh(),
        scratch_types=[
            pltpu.VMEM((_CH,), jnp.int32),
            pltpu.VMEM((_CH,), jnp.int32),
            pltpu.VMEM((_CH,), jnp.int32),
            pltpu.VMEM((_CH,), jnp.int32),
            pltpu.VMEM((_CH, _D), jnp.float32),
            pltpu.VMEM((_CH, _D), jnp.float32),
            pltpu.VMEM_SHARED((_NPAD, _D), jnp.float32),
            pltpu.SemaphoreType.DMA,
            pltpu.SemaphoreType.DMA,
        ],
    )
    def k(y_hbm, src_hbm, dst_hbm, zeros_hbm, out_hbm,
          sidx0, sidx1, didx0, didx1, rows0, rows1, acc, gsem0, gsem1):
        c = lax.axis_index("c")
        s = lax.axis_index("s")
        wid = s * _NC + c

        pltpu.sync_copy(zeros_hbm, acc.at[pl.ds(s * _RPS, _RPS)])

        plsc.subcore_barrier()

        def fire(ci, sidx, rows, gsem):
            pltpu.sync_copy(src_hbm.at[pl.ds(wid * _EPW + ci * _CH, _CH)], sidx)
            pltpu.async_copy(y_hbm.at[sidx], rows, gsem)

        def drain(sidx, rows, gsem):
            pltpu.make_async_copy(y_hbm.at[sidx], rows, gsem).wait()

        def scat(ci, didx, rows):
            pltpu.sync_copy(dst_hbm.at[pl.ds(wid * _EPW + ci * _CH, _CH)], didx)
            pltpu.sync_copy(rows, acc.at[didx], add=True)

        fire(0, sidx0, rows0, gsem0)
        fire(1, sidx1, rows1, gsem1)

        @pl.loop(0, _NCHUNK - 2, step=2)
        def _(ci):
            drain(sidx0, rows0, gsem0)
            scat(ci, didx0, rows0)
            fire(ci + 2, sidx0, rows0, gsem0)
            drain(sidx1, rows1, gsem1)
            scat(ci + 1, didx1, rows1)
            fire(ci + 3, sidx1, rows1, gsem1)

        drain(sidx0, rows0, gsem0)
        scat(_NCHUNK - 2, didx0, rows0)
        drain(sidx1, rows1, gsem1)
        scat(_NCHUNK - 1, didx1, rows1)

        plsc.subcore_barrier()

        @pl.loop(0, _RPS, step=_CH)
        def _(r0):
            r = s * _RPS + r0
            pltpu.sync_copy(acc.at[pl.ds(r, _CH)],
                            out_hbm.at[pl.ds(c * _NPAD + r, _CH)])

    return k(y, src, dst, zeros_hbm_v)


def _dis_expr(degp_ref):
    deg = degp_ref[0:_N, 0:1] + degp_ref[_NPAD:_NPAD + _N, 0:1] + 1.0
    return lax.rsqrt(deg)


def _tc_matmul(x, w):
    def body(x_ref, w_ref, o_ref):
        o_ref[...] = jnp.dot(x_ref[...], w_ref[...],
                             preferred_element_type=jnp.float32)

    return pl.pallas_call(
        body,
        out_shape=jax.ShapeDtypeStruct((x.shape[0], w.shape[1]), jnp.float32),
    )(x, w)


def _tc_scale(degp, xw):
    def body(degp_ref, xw_ref, o_ref):
        o_ref[...] = xw_ref[...] * _dis_expr(degp_ref)

    return pl.pallas_call(
        body,
        out_shape=jax.ShapeDtypeStruct((_N, _D), jnp.float32),
    )(degp, xw)


def _tc_mid(accp, y1, degp, b1, w2):
    def body(acc_ref, y1_ref, degp_ref, b1_ref, w2_ref, o_ref):
        dis = _dis_expr(degp_ref)
        ssum = acc_ref[0:_N, :] + acc_ref[_NPAD:_NPAD + _N, :] + y1_ref[...]
        h = jnp.maximum(ssum * dis + b1_ref[...], 0.0)
        o_ref[...] = jnp.dot(h, w2_ref[...],
                             preferred_element_type=jnp.float32) * dis

    return pl.pallas_call(
        body,
        out_shape=jax.ShapeDtypeStruct((_N, _D), jnp.float32),
    )(accp, y1, degp, b1, w2)


def _tc_out(accp, y2, degp, b2):
    def body(acc_ref, y2_ref, degp_ref, b2_ref, o_ref):
        dis = _dis_expr(degp_ref)
        z = (acc_ref[0:_N, :] + acc_ref[_NPAD:_NPAD + _N, :] + y2_ref[...]) \
            * dis + b2_ref[...]
        m = jnp.max(z, axis=1, keepdims=True)
        e = jnp.exp(z - m)
        o_ref[...] = (z - m) - jnp.log(jnp.sum(e, axis=1, keepdims=True))

    return pl.pallas_call(
        body,
        out_shape=jax.ShapeDtypeStruct((_N, _D), jnp.float32),
    )(accp, y2, degp, b2)


def kernel(feature, edge_index, use_feature, W1, b1, W2, b2):
    x = jnp.where(use_feature != 0, feature,
                  jnp.eye(_N, _D, dtype=feature.dtype))
    ei = edge_index.astype(jnp.int32)
    padw = _EPW - _E // _NW
    srcr = ei[0].reshape(_NW, _E // _NW)
    dstr = ei[1].reshape(_NW, _E // _NW)
    src = jnp.concatenate([srcr, jnp.zeros((_NW, padw), jnp.int32)],
                          axis=1).reshape(-1)
    trash = jnp.broadcast_to(_N + jnp.arange(padw, dtype=jnp.int32),
                             (_NW, padw))
    dst = jnp.concatenate([dstr, trash], axis=1).reshape(-1)
    b1r = b1.reshape(1, _D).astype(jnp.float32)
    b2r = b2.reshape(1, _D).astype(jnp.float32)
    ones_deg = jnp.ones((_CH, _DEGW), jnp.float32)
    zeros_deg = jnp.zeros((_RPS, _DEGW), jnp.float32)
    zeros_acc = jnp.zeros((_RPS, _D), jnp.float32)

    degp = _sc_degree(dst, ones_deg, zeros_deg)
    xw1 = _tc_matmul(x, W1)
    y1 = _tc_scale(degp, xw1)
    acc1 = _sc_scatter_accum(y1, src, dst, zeros_acc)
    y2 = _tc_mid(acc1, y1, degp, b1r, W2)
    acc2 = _sc_scatter_accum(y2, src, dst, zeros_acc)
    return _tc_out(acc2, y2, degp, b2r)
